# baseline (speedup 1.0000x reference)
import functools

import jax
import jax.numpy as jnp
from jax import lax
from jax.experimental import pallas as pl
from jax.experimental.pallas import tpu as pltpu
from jax.experimental.pallas import tpu_sc as plsc

_NC, _NS, _L = 2, 16, 16
_NW = _NC * _NS
_CH = 128


@functools.lru_cache(maxsize=None)
def _build(batch, emb_dim):
    bw = batch // _NW
    nch = bw // _CH
    mesh = plsc.VectorSubcoreMesh(
        core_axis_name="c", subcore_axis_name="s",
        num_cores=_NC, num_subcores=_NS)

    @functools.partial(
        pl.kernel,
        out_type=jax.ShapeDtypeStruct((2 * batch, emb_dim), jnp.float32),
        mesh=mesh,
        compiler_params=pltpu.CompilerParams(use_tc_tiling_on_sc=False),
        scratch_types=[
            pltpu.VMEM((nch, _CH), jnp.int32),
            pltpu.VMEM((nch, _CH), jnp.int32),
            pltpu.VMEM((nch, _CH), jnp.int32),
            pltpu.VMEM((nch, _CH), jnp.int32),
            pltpu.VMEM((bw, emb_dim), jnp.float32),
            pltpu.VMEM((bw, emb_dim), jnp.float32),
            pltpu.SemaphoreType.DMA,
            pltpu.SemaphoreType.DMA,
        ],
    )
    def lookup(uids_hbm, iids_hbm, uid_hbm, iid_hbm, out_hbm,
               uidx, iidx, oeidx, ooidx, urows, irows, gsem, ssem):
        wid = lax.axis_index("s") * _NC + lax.axis_index("c")
        base = wid * bw
        for c in range(nch):
            pltpu.sync_copy(uids_hbm.at[pl.ds(base + c * _CH, _CH)], uidx.at[c])
            pltpu.sync_copy(iids_hbm.at[pl.ds(base + c * _CH, _CH)], iidx.at[c])

        iota = lax.iota(jnp.int32, _L)
        for j in range(bw // _L):
            c, o = j // (_CH // _L), (j % (_CH // _L)) * _L
            oe = 2 * (base + j * _L + iota)
            oeidx[c, pl.ds(o, _L)] = oe
            ooidx[c, pl.ds(o, _L)] = oe + 1

        gathers = []
        for c in range(nch):
            gathers.append(pltpu.async_copy(
                uid_hbm.at[uidx.at[c]], urows.at[pl.ds(c * _CH, _CH)], gsem))
            gathers.append(pltpu.async_copy(
                iid_hbm.at[iidx.at[c]], irows.at[pl.ds(c * _CH, _CH)], gsem))
        for g in gathers:
            g.wait()

        scatters = []
        for c in range(nch):
            scatters.append(pltpu.async_copy(
                urows.at[pl.ds(c * _CH, _CH)], out_hbm.at[oeidx.at[c]], ssem))
            scatters.append(pltpu.async_copy(
                irows.at[pl.ds(c * _CH, _CH)], out_hbm.at[ooidx.at[c]], ssem))
        for s in scatters:
            s.wait()

    return lookup


def kernel(x, uid_table, iid_table):
    batch = x.shape[0]
    emb_dim = uid_table.shape[1]
    out = _build(batch, emb_dim)(x[:, 0], x[:, 1], uid_table, iid_table)
    return out.reshape(batch, 2, emb_dim)


# speedup vs baseline: 3.5285x; 3.5285x over previous
import functools

import jax
import jax.numpy as jnp
from jax import lax
from jax.experimental import pallas as pl
from jax.experimental.pallas import tpu as pltpu
from jax.experimental.pallas import tpu_sc as plsc

_NC, _NS, _L = 2, 16, 16
_NW = _NC * _NS
_NSLOT = 16


@functools.lru_cache(maxsize=None)
def _build(batch, emb_dim):
    bw = batch // _NW
    nblk = bw // _NSLOT
    mesh = plsc.VectorSubcoreMesh(
        core_axis_name="c", subcore_axis_name="s",
        num_cores=_NC, num_subcores=_NS)

    slot_types = [pltpu.VMEM((emb_dim, 128), jnp.float32)] * _NSLOT
    sem_types = [pltpu.SemaphoreType.DMA] * _NSLOT

    @functools.partial(
        pl.kernel,
        out_type=jax.ShapeDtypeStruct((2, emb_dim, batch), jnp.float32),
        mesh=mesh,
        compiler_params=pltpu.CompilerParams(
            use_tc_tiling_on_sc=True, needs_layout_passes=False),
        scratch_types=[
            pltpu.VMEM((2, bw), jnp.int32),
            pltpu.VMEM((emb_dim, bw), jnp.float32),
            pltpu.VMEM((emb_dim, bw), jnp.float32),
            pltpu.SemaphoreType.DMA,
        ] + slot_types + sem_types,
    )
    def lookup(xT_hbm, uT_hbm, iT_hbm, out_hbm, idv, uslab, islab, osem,
               *slots_and_sems):
        slots = slots_and_sems[:_NSLOT]
        sems = slots_and_sems[_NSLOT:]
        wid = lax.axis_index("s") * _NC + lax.axis_index("c")
        base = wid * bw
        pltpu.sync_copy(xT_hbm.at[:, pl.ds(base, bw)], idv)

        iota = lax.iota(jnp.int32, _L)
        iota_hi = iota + _L if emb_dim == 2 * _L else None

        def run_table(t_hbm, slab, s):
            def fetch_block(b):
                jv = idv[s, pl.ds(b * _NSLOT, _NSLOT)] & jnp.int32(~127)
                for k in range(_NSLOT):
                    pltpu.async_copy(
                        t_hbm.at[:, pl.ds(pl.multiple_of(jv[k], 128), 128)],
                        slots[k], sems[k])

            fetch_block(0)

            def body(b, carry):
                lv = idv[s, pl.ds(b * _NSLOT, _NSLOT)] & jnp.int32(127)
                for k in range(_NSLOT):
                    pltpu.make_async_copy(
                        t_hbm.at[:, pl.ds(0, 128)], slots[k], sems[k]).wait()
                    lane = jnp.broadcast_to(lv[k], (_L,))
                    pos = jnp.broadcast_to(b * _NSLOT + k, (_L,))
                    v0 = plsc.load_gather(slots[k], [iota, lane])
                    plsc.store_scatter(slab, [iota, pos], v0)
                    if iota_hi is not None:
                        v1 = plsc.load_gather(slots[k], [iota_hi, lane])
                        plsc.store_scatter(slab, [iota_hi, pos], v1)

                @pl.when(b + 1 < nblk)
                def _():
                    fetch_block(b + 1)

                return carry

            lax.fori_loop(0, nblk, body, 0)

        run_table(uT_hbm, uslab, 0)
        ocp0 = pltpu.async_copy(
            uslab, out_hbm.at[0].at[:, pl.ds(base, bw)], osem)
        run_table(iT_hbm, islab, 1)
        ocp1 = pltpu.async_copy(
            islab, out_hbm.at[1].at[:, pl.ds(base, bw)], osem)
        ocp0.wait()
        ocp1.wait()

    return lookup


def kernel(x, uid_table, iid_table):
    batch = x.shape[0]
    emb_dim = uid_table.shape[1]
    out3 = _build(batch, emb_dim)(x.T, uid_table.T, iid_table.T)
    return lax.transpose(out3, (2, 0, 1))


# speedup vs baseline: 3.5341x; 1.0016x over previous
import functools

import jax
import jax.numpy as jnp
from jax import lax
from jax.experimental import pallas as pl
from jax.experimental.pallas import tpu as pltpu
from jax.experimental.pallas import tpu_sc as plsc

_NC, _NS, _L = 2, 16, 16
_NW = _NC * _NS
_NSLOT = 16


@functools.lru_cache(maxsize=None)
def _build(batch, emb_dim):
    bw = batch // _NW
    nblk = bw // _NSLOT
    mesh = plsc.VectorSubcoreMesh(
        core_axis_name="c", subcore_axis_name="s",
        num_cores=_NC, num_subcores=_NS)

    slot_types = [pltpu.VMEM((emb_dim, 128), jnp.float32)] * _NSLOT
    sem_types = [pltpu.SemaphoreType.DMA] * _NSLOT

    @functools.partial(
        pl.kernel,
        out_type=jax.ShapeDtypeStruct((2, emb_dim, batch), jnp.float32),
        mesh=mesh,
        compiler_params=pltpu.CompilerParams(
            use_tc_tiling_on_sc=True, needs_layout_passes=False),
        scratch_types=[
            pltpu.VMEM((2, bw), jnp.int32),
            pltpu.VMEM((emb_dim, bw), jnp.float32),
            pltpu.VMEM((emb_dim, bw), jnp.float32),
            pltpu.SemaphoreType.DMA,
        ] + slot_types + sem_types,
    )
    def lookup(xT_hbm, uT_hbm, iT_hbm, out_hbm, idv, uslab, islab, osem,
               *slots_and_sems):
        slots = slots_and_sems[:_NSLOT]
        sems = slots_and_sems[_NSLOT:]
        wid = lax.axis_index("s") * _NC + lax.axis_index("c")
        base = wid * bw
        pltpu.sync_copy(xT_hbm.at[:, pl.ds(base, bw)], idv)

        iota = lax.iota(jnp.int32, _L)
        iota_hi = iota + _L if emb_dim == 2 * _L else None

        def run_table(t_hbm, slab, s):
            def window_starts(b):
                return idv[s, pl.ds(b * _NSLOT, _NSLOT)] & jnp.int32(~127)

            def fetch_block(b):
                jv = window_starts(b)
                for k in range(_NSLOT):
                    pltpu.async_copy(
                        t_hbm.at[:, pl.ds(pl.multiple_of(jv[k], 128), 128)],
                        slots[k], sems[k])

            fetch_block(0)

            def body(b, carry):
                lv = idv[s, pl.ds(b * _NSLOT, _NSLOT)] - window_starts(b)
                for k in range(_NSLOT):
                    pltpu.make_async_copy(
                        t_hbm.at[:, pl.ds(0, 128)], slots[k], sems[k]).wait()
                    lane = jnp.broadcast_to(lv[k], (_L,))
                    pos = jnp.broadcast_to(b * _NSLOT + k, (_L,))
                    v0 = plsc.load_gather(slots[k], [iota, lane])
                    plsc.store_scatter(slab, [iota, pos], v0)
                    if iota_hi is not None:
                        v1 = plsc.load_gather(slots[k], [iota_hi, lane])
                        plsc.store_scatter(slab, [iota_hi, pos], v1)

                @pl.when(b + 1 < nblk)
                def _():
                    fetch_block(b + 1)

                return carry

            lax.fori_loop(0, nblk, body, 0)

        run_table(uT_hbm, uslab, 0)
        ocp0 = pltpu.async_copy(
            uslab, out_hbm.at[0].at[:, pl.ds(base, bw)], osem)
        run_table(iT_hbm, islab, 1)
        ocp1 = pltpu.async_copy(
            islab, out_hbm.at[1].at[:, pl.ds(base, bw)], osem)
        ocp0.wait()
        ocp1.wait()

    return lookup


def kernel(x, uid_table, iid_table):
    batch = x.shape[0]
    emb_dim = uid_table.shape[1]
    out3 = _build(batch, emb_dim)(x.T, uid_table.T, iid_table.T)
    return lax.transpose(out3, (2, 0, 1))
